# baseline (speedup 1.0000x reference)
import functools

import jax
import jax.numpy as jnp
from jax import lax
from jax.experimental import pallas as pl
from jax.experimental.pallas import tpu as pltpu
from jax.experimental.pallas import tpu_sc as plsc

_MESH = plsc.VectorSubcoreMesh(
    core_axis_name="c", subcore_axis_name="s", num_cores=1, num_subcores=1
)


@functools.partial(
    pl.kernel,
    out_type=jax.ShapeDtypeStruct((4,), jnp.float32),
    mesh=_MESH,
    scratch_types=[
        pltpu.VMEM((16,), jnp.float32),
        pltpu.VMEM((16,), jnp.float32),
    ],
    compiler_params=pltpu.CompilerParams(
        needs_layout_passes=False, skip_device_barrier=True
    ),
)
def _gather_sc(x_hbm, out_hbm, buf_v, res_v):
    pltpu.sync_copy(x_hbm.at[pl.ds(0, 16)], buf_v)
    lane = lax.iota(jnp.int32, 16)
    idx = jnp.where(
        lane == 0,
        2,
        jnp.where(lane == 1, 0, jnp.where(lane == 2, 4, jnp.where(lane == 3, 1, 0))),
    )
    res_v[...] = plsc.load_gather(buf_v, [idx])
    pltpu.sync_copy(res_v.at[pl.ds(0, 4)], out_hbm)


def kernel(x):
    return _gather_sc(x)


# speedup vs baseline: 13.4881x; 13.4881x over previous
import jax
import jax.numpy as jnp
from jax.experimental import pallas as pl
from jax.experimental.pallas import tpu as pltpu


def _body(x_ref, o_ref):
    o_ref[0] = x_ref[2]
    o_ref[1] = x_ref[0]
    o_ref[2] = x_ref[4]
    o_ref[3] = x_ref[1]


def kernel(x):
    return pl.pallas_call(
        _body,
        out_shape=jax.ShapeDtypeStruct((4,), jnp.float32),
        grid=(1,),
        in_specs=[pl.BlockSpec((128,), lambda i: (0,), memory_space=pltpu.SMEM)],
        out_specs=pl.BlockSpec((4,), lambda i: (0,), memory_space=pltpu.SMEM),
    )(x)
